# baseline (device time: 64884 ns/iter reference)
import jax
import jax.numpy as jnp
from jax import lax
from jax.experimental import pallas as pl
from jax.experimental.pallas import tpu as pltpu

N_DEV = 4
B, SQ, SKV, HQ, DH = 2, 512, 512, 8, 64
DM = 768
DQ = HQ * DH
ROWS = B * SQ
CH = ROWS // N_DEV


def kernel(x, Wq, K_ext, V_ext, Wo):
    x2 = x.reshape(ROWS, DM)

    def body(x_ref, wq_ref, k_hbm, v_hbm, wo_ref, out_ref,
             kt_ref, vt_ref, ctx_ref, part_ref, rs_ref, agsrc_ref, ag_ref,
             k_sems, v_sems, send_sems, rs_recv_sems, ag_recv_sems):
        me = lax.axis_index("i")

        def kv_copies(b):
            for h in range(HQ):
                bh = b * HQ + h
                hh = me * HQ + h
                yield pltpu.make_async_copy(
                    k_hbm.at[b, :, hh, :], kt_ref.at[bh], k_sems.at[bh])
                yield pltpu.make_async_copy(
                    v_hbm.at[b, :, hh, :], vt_ref.at[bh], v_sems.at[bh])

        for b in range(B):
            for cp in kv_copies(b):
                cp.start()

        barrier_sem = pltpu.get_barrier_semaphore()
        for r in range(1, N_DEV):
            pl.semaphore_signal(barrier_sem, inc=1,
                                device_id=(lax.rem(me + r, N_DEV),),
                                device_id_type=pl.DeviceIdType.MESH)
        pl.semaphore_wait(barrier_sem, N_DEV - 1)

        ki = lax.broadcasted_iota(jnp.int32, (CH, SKV), 1)
        qi0 = lax.broadcasted_iota(jnp.int32, (CH, SKV), 0)

        def compute_chunk(c, wait_pred):
            b = lax.div(c, 2)
            qoff = lax.rem(c, 2) * CH
            xq = x_ref[pl.ds(c * CH, CH), :]
            q_c = jnp.dot(xq, wq_ref[:],
                          preferred_element_type=jnp.float32) * 0.125

            @pl.when(wait_pred)
            def _():
                for cp in kv_copies(b):
                    cp.wait()

            qi = qi0 + qoff
            mask = (jnp.abs(qi - ki) <= 128) | (ki < 32) | (qi < 32)
            for h in range(HQ):
                bh = b * HQ + h
                q = q_c[:, h * DH:(h + 1) * DH]
                k = kt_ref[bh]
                s = lax.dot_general(q, k, (((1,), (1,)), ((), ())),
                                    preferred_element_type=jnp.float32)
                w = jnp.exp(jnp.where(mask, s, -1e9))
                ctx = jnp.dot(w, vt_ref[bh],
                              preferred_element_type=jnp.float32)
                ctx = ctx / jnp.sum(w, axis=1, keepdims=True)
                ctx_ref[:, h * DH:(h + 1) * DH] = ctx
            return jnp.dot(ctx_ref[:], wo_ref[:],
                           preferred_element_type=jnp.float32)

        cs = [lax.rem(me + i + 1, N_DEV) for i in range(N_DEV)]
        bs = [lax.div(c, 2) for c in cs]
        waits = [jnp.bool_(True), bs[1] != bs[0], bs[0] == bs[1],
                 jnp.bool_(False)]

        rs = []
        for r in range(1, N_DEV):
            c = cs[r - 1]
            part_ref[r - 1, :, :] = compute_chunk(
                c, waits[r - 1]).astype(jnp.bfloat16)
            rdma = pltpu.make_async_remote_copy(
                src_ref=part_ref.at[r - 1],
                dst_ref=rs_ref.at[r - 1],
                send_sem=send_sems.at[r - 1],
                recv_sem=rs_recv_sems.at[r - 1],
                device_id=(c,),
                device_id_type=pl.DeviceIdType.MESH,
            )
            rdma.start()
            rs.append(rdma)

        own = compute_chunk(cs[3], waits[3])
        for rdma in rs:
            rdma.wait_recv()
        red = (own
               + rs_ref[0].astype(jnp.float32)
               + rs_ref[1].astype(jnp.float32)
               + rs_ref[2].astype(jnp.float32))
        out_ref[pl.ds(me * CH, CH), :] = red
        agsrc_ref[:] = red.astype(jnp.bfloat16)
        for rdma in rs:
            rdma.wait_send()

        ag = []
        for r in range(1, N_DEV):
            p = lax.rem(me + r, N_DEV)
            rdma = pltpu.make_async_remote_copy(
                src_ref=agsrc_ref,
                dst_ref=ag_ref.at[N_DEV - 1 - r],
                send_sem=send_sems.at[r - 1],
                recv_sem=ag_recv_sems.at[N_DEV - 1 - r],
                device_id=(p,),
                device_id_type=pl.DeviceIdType.MESH,
            )
            rdma.start()
            ag.append(rdma)
        for rdma in ag:
            rdma.wait_recv()
        for j in range(N_DEV - 1):
            p = lax.rem(me + j + 1, N_DEV)
            out_ref[pl.ds(p * CH, CH), :] = ag_ref[j].astype(jnp.float32)
        for rdma in ag:
            rdma.wait_send()

    out = pl.pallas_call(
        body,
        out_shape=jax.ShapeDtypeStruct((ROWS, DM), jnp.float32),
        in_specs=[
            pl.BlockSpec(memory_space=pltpu.VMEM),
            pl.BlockSpec(memory_space=pltpu.VMEM),
            pl.BlockSpec(memory_space=pl.ANY),
            pl.BlockSpec(memory_space=pl.ANY),
            pl.BlockSpec(memory_space=pltpu.VMEM),
        ],
        out_specs=pl.BlockSpec(memory_space=pltpu.VMEM),
        scratch_shapes=[
            pltpu.VMEM((B * HQ, SKV, DH), jnp.float32),
            pltpu.VMEM((B * HQ, SKV, DH), jnp.float32),
            pltpu.VMEM((CH, DQ), jnp.float32),
            pltpu.VMEM((N_DEV - 1, CH, DM), jnp.bfloat16),
            pltpu.VMEM((N_DEV - 1, CH, DM), jnp.bfloat16),
            pltpu.VMEM((CH, DM), jnp.bfloat16),
            pltpu.VMEM((N_DEV - 1, CH, DM), jnp.bfloat16),
            pltpu.SemaphoreType.DMA((B * HQ,)),
            pltpu.SemaphoreType.DMA((B * HQ,)),
            pltpu.SemaphoreType.DMA((N_DEV - 1,)),
            pltpu.SemaphoreType.DMA((N_DEV - 1,)),
            pltpu.SemaphoreType.DMA((N_DEV - 1,)),
        ],
        compiler_params=pltpu.CompilerParams(collective_id=0),
    )(x2, Wq, K_ext, V_ext, Wo)
    return out.reshape(B, SQ, DM)


# device time: 34561 ns/iter; 1.8774x vs baseline; 1.8774x over previous
import jax
import jax.numpy as jnp
from jax import lax
from jax.experimental import pallas as pl
from jax.experimental.pallas import tpu as pltpu

N_DEV = 4
B, SQ, SKV, HQ, DH = 2, 512, 512, 8, 64
DM = 768
DQ = HQ * DH
ROWS = B * SQ
CH = ROWS // N_DEV


def kernel(x, Wq, K_ext, V_ext, Wo):
    my = lax.axis_index("i")
    K = lax.dynamic_slice_in_dim(K_ext, my * HQ, HQ, axis=2)
    V = lax.dynamic_slice_in_dim(V_ext, my * HQ, HQ, axis=2)
    bf = jnp.bfloat16
    Kt = jnp.transpose(K.astype(bf), (0, 2, 1, 3)).reshape(B * HQ, SKV, DH)
    Vt = jnp.transpose(V.astype(bf), (0, 2, 1, 3)).reshape(B * HQ, SKV, DH)
    x2 = x.reshape(ROWS, DM)

    def body(x_ref, wq_ref, kt_ref, vt_ref, wo_ref, out_ref,
             ctx_ref, part_ref, rs_ref, agsrc_ref, ag_ref,
             send_sems, rs_recv_sems, ag_send_sems, ag_recv_sems):
        me = lax.axis_index("i")

        barrier_sem = pltpu.get_barrier_semaphore()
        for r in range(1, N_DEV):
            pl.semaphore_signal(barrier_sem, inc=1,
                                device_id=(lax.rem(me + r, N_DEV),),
                                device_id_type=pl.DeviceIdType.MESH)
        pl.semaphore_wait(barrier_sem, N_DEV - 1)

        ki = lax.broadcasted_iota(jnp.int32, (CH, SKV), 1)
        qi0 = lax.broadcasted_iota(jnp.int32, (CH, SKV), 0)

        def compute_chunk(c):
            b = lax.div(c, 2)
            qoff = lax.rem(c, 2) * CH
            xq = x_ref[pl.ds(c * CH, CH), :]
            q_c = jnp.dot(xq, wq_ref[:],
                          preferred_element_type=jnp.float32) * 0.125
            qi = qi0 + qoff
            mask = (jnp.abs(qi - ki) <= 128) | (ki < 32) | (qi < 32)
            for h in range(HQ):
                bh = b * HQ + h
                q = q_c[:, h * DH:(h + 1) * DH]
                k = kt_ref[bh].astype(jnp.float32)
                s = lax.dot_general(q, k, (((1,), (1,)), ((), ())),
                                    preferred_element_type=jnp.float32)
                w = jnp.exp(jnp.where(mask, s, -1e9))
                ctx = jnp.dot(w, vt_ref[bh].astype(jnp.float32),
                              preferred_element_type=jnp.float32)
                ctx = ctx / jnp.sum(w, axis=1, keepdims=True)
                ctx_ref[:, h * DH:(h + 1) * DH] = ctx
            return jnp.dot(ctx_ref[:], wo_ref[:],
                           preferred_element_type=jnp.float32)

        HDM = DM // 2

        rs = [[], []]
        for r in range(1, N_DEV):
            c = lax.rem(me + r, N_DEV)
            part_ref[r - 1, :, :] = compute_chunk(c).astype(jnp.bfloat16)
            for hf in range(2):
                rdma = pltpu.make_async_remote_copy(
                    src_ref=part_ref.at[r - 1, :, pl.ds(hf * HDM, HDM)],
                    dst_ref=rs_ref.at[r - 1, :, pl.ds(hf * HDM, HDM)],
                    send_sem=send_sems.at[(r - 1) * 2 + hf],
                    recv_sem=rs_recv_sems.at[(r - 1) * 2 + hf],
                    device_id=(c,),
                    device_id_type=pl.DeviceIdType.MESH,
                )
                rdma.start()
                rs[hf].append(rdma)

        own = compute_chunk(me)

        ag = []
        for hf in range(2):
            for rdma in rs[hf]:
                rdma.wait_recv()
            cols = pl.ds(hf * HDM, HDM)
            redh = (own[:, hf * HDM:(hf + 1) * HDM]
                    + rs_ref[0, :, cols].astype(jnp.float32)
                    + rs_ref[1, :, cols].astype(jnp.float32)
                    + rs_ref[2, :, cols].astype(jnp.float32))
            out_ref[pl.ds(me * CH, CH), cols] = redh
            agsrc_ref[:, cols] = redh.astype(jnp.bfloat16)
            for r in range(1, N_DEV):
                p = lax.rem(me + r, N_DEV)
                rdma = pltpu.make_async_remote_copy(
                    src_ref=agsrc_ref.at[:, cols],
                    dst_ref=ag_ref.at[N_DEV - 1 - r, :, cols],
                    send_sem=ag_send_sems.at[(r - 1) * 2 + hf],
                    recv_sem=ag_recv_sems.at[(N_DEV - 1 - r) * 2 + hf],
                    device_id=(p,),
                    device_id_type=pl.DeviceIdType.MESH,
                )
                rdma.start()
                ag.append(rdma)

        for rdma in ag:
            rdma.wait_recv()
        for j in range(N_DEV - 1):
            p = lax.rem(me + j + 1, N_DEV)
            out_ref[pl.ds(p * CH, CH), :] = ag_ref[j].astype(jnp.float32)
        for hf in range(2):
            for rdma in rs[hf]:
                rdma.wait_send()
        for rdma in ag:
            rdma.wait_send()

    out = pl.pallas_call(
        body,
        out_shape=jax.ShapeDtypeStruct((ROWS, DM), jnp.float32),
        in_specs=[pl.BlockSpec(memory_space=pltpu.VMEM)] * 5,
        out_specs=pl.BlockSpec(memory_space=pltpu.VMEM),
        scratch_shapes=[
            pltpu.VMEM((CH, DQ), jnp.float32),
            pltpu.VMEM((N_DEV - 1, CH, DM), jnp.bfloat16),
            pltpu.VMEM((N_DEV - 1, CH, DM), jnp.bfloat16),
            pltpu.VMEM((CH, DM), jnp.bfloat16),
            pltpu.VMEM((N_DEV - 1, CH, DM), jnp.bfloat16),
            pltpu.SemaphoreType.DMA((2 * (N_DEV - 1),)),
            pltpu.SemaphoreType.DMA((2 * (N_DEV - 1),)),
            pltpu.SemaphoreType.DMA((2 * (N_DEV - 1),)),
            pltpu.SemaphoreType.DMA((2 * (N_DEV - 1),)),
        ],
        compiler_params=pltpu.CompilerParams(collective_id=0),
    )(x2, Wq, Kt, Vt, Wo)
    return out.reshape(B, SQ, DM)
